# baseline (speedup 1.0000x reference)
import jax
import jax.numpy as jnp
from jax.experimental import pallas as pl
from jax.experimental.pallas import tpu as pltpu

NUM_EXPERT = 64
IN_FEAT = 768
OUT_FEAT = 768
N_TOKENS = 128
EPG = 4
NSTEPS = NUM_EXPERT // EPG


def _moe_kernel(ids_ref, nu_ref, gate_ref, inp_ref, w0, w1, w2, w3, out_ref):
    s = pl.program_id(0)

    @pl.when(s == 0)
    def _init():
        out_ref[...] = jnp.zeros_like(out_ref)

    n_used = nu_ref[0]
    for j, w in enumerate((w0, w1, w2, w3)):
        pos = s * EPG + j

        @pl.when(pos < n_used)
        def _acc(w=w, pos=pos):
            e = ids_ref[pos]
            mask = gate_ref[...] == e
            x = jnp.where(mask, inp_ref[...], 0.0)
            out_ref[...] += jax.lax.dot_general(
                x, w[0],
                (((1,), (1,)), ((), ())),
                preferred_element_type=jnp.float32,
            )


def _routing_metadata(gate):
    sg = jnp.sort(gate)
    is_new = jnp.concatenate(
        [jnp.ones((1,), jnp.bool_), sg[1:] != sg[:-1]]
    )
    n_used = jnp.sum(is_new.astype(jnp.int32)).reshape(1)
    rank = jnp.cumsum(is_new.astype(jnp.int32)) - 1
    ids = jnp.zeros((NUM_EXPERT,), jnp.int32).at[rank].set(sg)
    ids = jnp.where(jnp.arange(NUM_EXPERT) < n_used[0], ids, sg[-1])
    return ids, n_used


def kernel(inp, gate, weight):
    gate2d = gate.reshape(N_TOKENS, 1)
    ids, n_used = _routing_metadata(gate)
    w_specs = [
        pl.BlockSpec(
            (1, OUT_FEAT, IN_FEAT),
            lambda s, ids_ref, nu_ref, jj=j: (ids_ref[EPG * s + jj], 0, 0),
        )
        for j in range(EPG)
    ]
    grid_spec = pltpu.PrefetchScalarGridSpec(
        num_scalar_prefetch=2,
        grid=(NSTEPS,),
        in_specs=[
            pl.BlockSpec((N_TOKENS, 1), lambda s, ids_ref, nu_ref: (0, 0)),
            pl.BlockSpec((N_TOKENS, IN_FEAT), lambda s, ids_ref, nu_ref: (0, 0)),
        ] + w_specs,
        out_specs=pl.BlockSpec(
            (N_TOKENS, OUT_FEAT), lambda s, ids_ref, nu_ref: (0, 0)
        ),
    )
    return pl.pallas_call(
        _moe_kernel,
        grid_spec=grid_spec,
        out_shape=jax.ShapeDtypeStruct((N_TOKENS, OUT_FEAT), jnp.float32),
    )(ids, n_used, gate2d, inp, weight, weight, weight, weight)


# speedup vs baseline: 1.1749x; 1.1749x over previous
import jax
import jax.numpy as jnp
from jax.experimental import pallas as pl

NUM_EXPERT = 64
IN_FEAT = 768
OUT_FEAT = 768
N_TOKENS = 128
EPG = 4
NSTEPS = NUM_EXPERT // EPG


def _moe_kernel(gate_ref, inp_ref, w_ref, out_ref):
    s = pl.program_id(0)

    @pl.when(s == 0)
    def _init():
        out_ref[...] = jnp.zeros_like(out_ref)

    acc = out_ref[...]
    for j in range(EPG):
        e = s * EPG + j
        mask = gate_ref[...] == e
        x = jnp.where(mask, inp_ref[...], 0.0)
        acc += jax.lax.dot_general(
            x, w_ref[j],
            (((1,), (1,)), ((), ())),
            preferred_element_type=jnp.float32,
            precision=jax.lax.Precision.DEFAULT,
        )
    out_ref[...] = acc


def kernel(inp, gate, weight):
    gate2d = gate.reshape(N_TOKENS, 1)
    return pl.pallas_call(
        _moe_kernel,
        grid=(NSTEPS,),
        in_specs=[
            pl.BlockSpec((N_TOKENS, 1), lambda s: (0, 0)),
            pl.BlockSpec((N_TOKENS, IN_FEAT), lambda s: (0, 0)),
            pl.BlockSpec((EPG, OUT_FEAT, IN_FEAT), lambda s: (s, 0, 0)),
        ],
        out_specs=pl.BlockSpec((N_TOKENS, OUT_FEAT), lambda s: (0, 0)),
        out_shape=jax.ShapeDtypeStruct((N_TOKENS, OUT_FEAT), jnp.float32),
    )(gate2d, inp, weight)
